# Initial kernel scaffold; baseline (speedup 1.0000x reference)
import jax
import jax.numpy as jnp
from jax.experimental import pallas as pl


def kernel(x, adj_weight, out_proj):
    raise NotImplementedError("write your pallas kernel here")


# speedup vs baseline: 3.1548x; 3.1548x over previous
import jax
import jax.numpy as jnp
from jax.experimental import pallas as pl

SEG = 2048
K = 8
ROW_BLK = 256
M_BLK = 1024


def _weff_kernel(w_ref, op_ref, out_ref):
    w = w_ref[...]
    iota = jax.lax.broadcasted_iota(jnp.int32, w.shape, 1)
    work = w
    mask = jnp.zeros(w.shape, dtype=jnp.bool_)
    for _ in range(K):
        m = jnp.max(work, axis=1, keepdims=True)
        eligible = work == m
        idx = jnp.min(jnp.where(eligible, iota, SEG), axis=1, keepdims=True)
        sel = iota == idx
        mask = jnp.logical_or(mask, sel)
        work = jnp.where(sel, -jnp.inf, work)
    masked = jnp.where(mask, w, 0.0)
    row_max = jnp.max(masked, axis=1, keepdims=True)
    e = jnp.exp(masked - row_max)
    z = jnp.sum(e, axis=1, keepdims=True)
    adj = e / z
    out_ref[...] = jnp.dot(adj, op_ref[...], preferred_element_type=jnp.float32)


def _mm_kernel(x_ref, w_ref, o_ref):
    o_ref[...] = jnp.dot(x_ref[...], w_ref[...], preferred_element_type=jnp.float32)


def kernel(x, adj_weight, out_proj):
    B, T, S = x.shape
    w_eff = pl.pallas_call(
        _weff_kernel,
        grid=(SEG // ROW_BLK,),
        in_specs=[
            pl.BlockSpec((ROW_BLK, SEG), lambda i: (i, 0)),
            pl.BlockSpec((SEG, SEG), lambda i: (0, 0)),
        ],
        out_specs=pl.BlockSpec((ROW_BLK, SEG), lambda i: (i, 0)),
        out_shape=jax.ShapeDtypeStruct((SEG, SEG), jnp.float32),
    )(adj_weight, out_proj)
    xm = x.reshape(B * T, S)
    out = pl.pallas_call(
        _mm_kernel,
        grid=(B * T // M_BLK,),
        in_specs=[
            pl.BlockSpec((M_BLK, S), lambda i: (i, 0)),
            pl.BlockSpec((S, S), lambda i: (0, 0)),
        ],
        out_specs=pl.BlockSpec((M_BLK, S), lambda i: (i, 0)),
        out_shape=jax.ShapeDtypeStruct((B * T, S), jnp.float32),
    )(xm, w_eff)
    return out.reshape(B, T, S)


# speedup vs baseline: 3.2001x; 1.0144x over previous
import jax
import jax.numpy as jnp
from jax.experimental import pallas as pl

SEG = 2048
K = 8
ROW_BLK = 256
M_BLK = 1024


def _weff_kernel(w_ref, op_ref, out_ref):
    w = w_ref[...]
    iota = jax.lax.broadcasted_iota(jnp.int32, w.shape, 1)
    work = w
    mask = jnp.zeros(w.shape, dtype=jnp.bool_)
    for _ in range(K):
        m = jnp.max(work, axis=1, keepdims=True)
        eligible = work == m
        idx = jnp.min(jnp.where(eligible, iota, SEG), axis=1, keepdims=True)
        sel = iota == idx
        mask = jnp.logical_or(mask, sel)
        work = jnp.where(sel, -jnp.inf, work)
    masked = jnp.where(mask, w, 0.0)
    row_max = jnp.max(masked, axis=1, keepdims=True)
    e = jnp.exp(masked - row_max)
    z = jnp.sum(e, axis=1, keepdims=True)
    adj = (e / z).astype(jnp.bfloat16)
    op = op_ref[...].astype(jnp.bfloat16)
    out_ref[...] = jnp.dot(adj, op, preferred_element_type=jnp.float32)


def _mm_kernel(x_ref, w_ref, o_ref):
    xb = x_ref[...].astype(jnp.bfloat16)
    wb = w_ref[...].astype(jnp.bfloat16)
    o_ref[...] = jnp.dot(xb, wb, preferred_element_type=jnp.float32)


def kernel(x, adj_weight, out_proj):
    B, T, S = x.shape
    w_eff = pl.pallas_call(
        _weff_kernel,
        grid=(SEG // ROW_BLK,),
        in_specs=[
            pl.BlockSpec((ROW_BLK, SEG), lambda i: (i, 0)),
            pl.BlockSpec((SEG, SEG), lambda i: (0, 0)),
        ],
        out_specs=pl.BlockSpec((ROW_BLK, SEG), lambda i: (i, 0)),
        out_shape=jax.ShapeDtypeStruct((SEG, SEG), jnp.float32),
    )(adj_weight, out_proj)
    xm = x.reshape(B * T, S)
    out = pl.pallas_call(
        _mm_kernel,
        grid=(B * T // M_BLK,),
        in_specs=[
            pl.BlockSpec((M_BLK, S), lambda i: (i, 0)),
            pl.BlockSpec((S, S), lambda i: (0, 0)),
        ],
        out_specs=pl.BlockSpec((M_BLK, S), lambda i: (i, 0)),
        out_shape=jax.ShapeDtypeStruct((B * T, S), jnp.float32),
    )(xm, w_eff)
    return out.reshape(B, T, S)


# speedup vs baseline: 3.4742x; 1.0857x over previous
import jax
import jax.numpy as jnp
from jax.experimental import pallas as pl

SEG = 2048
K = 8
ROW_BLK = 256
M_BLK = 1024


def _weff_kernel(w_ref, op_ref, out_ref):
    w = w_ref[...]
    work = w
    mask = jnp.zeros(w.shape, dtype=jnp.bool_)
    row_max = None
    for i in range(K):
        m = jnp.max(work, axis=1, keepdims=True)
        if i == 0:
            row_max = jnp.maximum(m, 0.0)
        sel = work == m
        mask = jnp.logical_or(mask, sel)
        work = jnp.where(sel, -jnp.inf, work)
    masked = jnp.where(mask, w, 0.0)
    e = jnp.exp(masked - row_max)
    z = jnp.sum(e, axis=1, keepdims=True)
    adj = (e / z).astype(jnp.bfloat16)
    op = op_ref[...].astype(jnp.bfloat16)
    out_ref[...] = jnp.dot(adj, op, preferred_element_type=jnp.float32)


def _mm_kernel(x_ref, w_ref, o_ref):
    xb = x_ref[...].astype(jnp.bfloat16)
    wb = w_ref[...].astype(jnp.bfloat16)
    o_ref[...] = jnp.dot(xb, wb, preferred_element_type=jnp.float32)


def kernel(x, adj_weight, out_proj):
    B, T, S = x.shape
    w_eff = pl.pallas_call(
        _weff_kernel,
        grid=(SEG // ROW_BLK,),
        in_specs=[
            pl.BlockSpec((ROW_BLK, SEG), lambda i: (i, 0)),
            pl.BlockSpec((SEG, SEG), lambda i: (0, 0)),
        ],
        out_specs=pl.BlockSpec((ROW_BLK, SEG), lambda i: (i, 0)),
        out_shape=jax.ShapeDtypeStruct((SEG, SEG), jnp.float32),
    )(adj_weight, out_proj)
    xm = x.reshape(B * T, S)
    out = pl.pallas_call(
        _mm_kernel,
        grid=(B * T // M_BLK,),
        in_specs=[
            pl.BlockSpec((M_BLK, S), lambda i: (i, 0)),
            pl.BlockSpec((S, S), lambda i: (0, 0)),
        ],
        out_specs=pl.BlockSpec((M_BLK, S), lambda i: (i, 0)),
        out_shape=jax.ShapeDtypeStruct((B * T, S), jnp.float32),
    )(xm, w_eff)
    return out.reshape(B, T, S)
